# Initial kernel scaffold; baseline (speedup 1.0000x reference)
import jax
import jax.numpy as jnp
from jax.experimental import pallas as pl


def kernel(lookup_ids, embedding_matrix):
    raise NotImplementedError("write your pallas kernel here")


# speedup vs baseline: 1.4885x; 1.4885x over previous
import functools

import jax
import jax.numpy as jnp
from jax import lax
from jax.experimental import pallas as pl
from jax.experimental.pallas import tpu as pltpu
from jax.experimental.pallas import tpu_sc as plsc

B = 4096
L = 200
D = 32
TOT = B * L
NC = 2
NS = 16
NW = NC * NS
PER_W = TOT // NW
CHUNK = 1600
NCHUNK = PER_W // CHUNK

_mesh = plsc.VectorSubcoreMesh(core_axis_name="c", subcore_axis_name="s")


@functools.partial(
    pl.kernel,
    mesh=_mesh,
    compiler_params=pltpu.CompilerParams(use_tc_tiling_on_sc=False),
    out_type=jax.ShapeDtypeStruct((TOT, D), jnp.float32),
    scratch_types=[
        pltpu.VMEM((CHUNK,), jnp.int32),
        pltpu.VMEM((CHUNK,), jnp.int32),
        pltpu.VMEM((CHUNK, D), jnp.float32),
        pltpu.VMEM((CHUNK, D), jnp.float32),
        pltpu.SemaphoreType.DMA,
        pltpu.SemaphoreType.DMA,
    ],
)
def _gather_sc(idx_hbm, table_hbm, out_hbm, idx0, idx1, rows0, rows1,
               gsem0, gsem1):
    wid = lax.axis_index("s") * NC + lax.axis_index("c")
    base = wid * PER_W

    bufs = [(idx0, rows0, gsem0), (idx1, rows1, gsem1)]

    pltpu.sync_copy(idx_hbm.at[pl.ds(base, CHUNK)], idx0)
    pltpu.make_async_copy(table_hbm.at[idx0], rows0, gsem0).start()
    for i in range(NCHUNK):
        cidx, crows, csem = bufs[i % 2]
        nidx, nrows, nsem = bufs[(i + 1) % 2]
        if i + 1 < NCHUNK:
            pltpu.sync_copy(idx_hbm.at[pl.ds(base + (i + 1) * CHUNK, CHUNK)],
                            nidx)
            pltpu.make_async_copy(table_hbm.at[nidx], nrows, nsem).start()
        pltpu.make_async_copy(table_hbm.at[cidx], crows, csem).wait()
        pltpu.sync_copy(crows, out_hbm.at[pl.ds(base + i * CHUNK, CHUNK)])


def _mask_body(ids_ref, mask_ref):
    mask_ref[...] = ids_ref[...] == 0


_mask_tc = pl.pallas_call(
    _mask_body,
    out_shape=jax.ShapeDtypeStruct((B, L), jnp.bool_),
)


def kernel(lookup_ids, embedding_matrix):
    flat_ids = lookup_ids.reshape(-1)
    gathered = _gather_sc(flat_ids, embedding_matrix)
    matrices = gathered.reshape(B, L, D)
    mask = _mask_tc(lookup_ids)
    return matrices, mask, lookup_ids
